# Initial kernel scaffold; baseline (speedup 1.0000x reference)
import jax
import jax.numpy as jnp
from jax.experimental import pallas as pl


def kernel(constraint_features, edge_indices, edge_features, variable_features, params):
    raise NotImplementedError("write your pallas kernel here")


# speedup vs baseline: 1.9302x; 1.9302x over previous
import functools

import jax
import jax.numpy as jnp
from jax import lax
from jax.experimental import pallas as pl
from jax.experimental.pallas import tpu as pltpu
from jax.experimental.pallas import tpu_sc as plsc

EMB = 64
HALF = 32
LANES = 16
N_TILES = 16
N_SC = 2
BLK = 80
ROWS = 400


def _mlp2_body(x_ref, w1_ref, b1_ref, w2_ref, b2_ref, o_ref):
    h = jnp.maximum(
        jnp.dot(x_ref[...], w1_ref[...], preferred_element_type=jnp.float32,
                precision=lax.Precision.HIGHEST)
        + b1_ref[...], 0.0)
    o_ref[...] = jnp.maximum(
        jnp.dot(h, w2_ref[...], preferred_element_type=jnp.float32,
                precision=lax.Precision.HIGHEST)
        + b2_ref[...], 0.0)


def _mlp2(x, w1, b1, w2, b2):
    n, k = x.shape
    g = n // ROWS
    return pl.pallas_call(
        _mlp2_body,
        grid=(g,),
        in_specs=[
            pl.BlockSpec((ROWS, k), lambda i: (i, 0)),
            pl.BlockSpec((k, EMB), lambda i: (0, 0)),
            pl.BlockSpec((1, EMB), lambda i: (0, 0)),
            pl.BlockSpec((EMB, EMB), lambda i: (0, 0)),
            pl.BlockSpec((1, EMB), lambda i: (0, 0)),
        ],
        out_specs=pl.BlockSpec((ROWS, EMB), lambda i: (i, 0)),
        out_shape=jax.ShapeDtypeStruct((n, EMB), jnp.float32),
    )(x, w1, b1.reshape(1, EMB), w2, b2.reshape(1, EMB))


def _mm_body(x_ref, w_ref, o_ref):
    o_ref[...] = jnp.dot(x_ref[...], w_ref[0],
                         preferred_element_type=jnp.float32,
                precision=lax.Precision.HIGHEST)


def _matmul_packed(x, w):
    n = x.shape[0]
    g = n // ROWS
    ws = jnp.stack([w[:, :HALF], w[:, HALF:]])
    return pl.pallas_call(
        _mm_body,
        grid=(2, g),
        in_specs=[
            pl.BlockSpec((ROWS, EMB), lambda h, i: (i, 0)),
            pl.BlockSpec((1, EMB, HALF), lambda h, i: (h, 0, 0)),
        ],
        out_specs=pl.BlockSpec((ROWS, HALF), lambda h, i: (h * g + i, 0)),
        out_shape=jax.ShapeDtypeStruct((2 * n, HALF), jnp.float32),
    )(x, ws)


def _node_core(s0_ref, s1_ref, cnt_ref, rt_ref, g0_ref, g1_ref, gb_ref,
               wr_ref, bt_ref, o2w_ref, o2b_ref):
    x = (jnp.dot(s0_ref[...], g0_ref[...], preferred_element_type=jnp.float32,
                precision=lax.Precision.HIGHEST)
         + jnp.dot(s1_ref[...], g1_ref[...], preferred_element_type=jnp.float32,
                precision=lax.Precision.HIGHEST)
         + cnt_ref[:, 0:1] * gb_ref[...]
         + jnp.dot(rt_ref[...], wr_ref[...], preferred_element_type=jnp.float32,
                precision=lax.Precision.HIGHEST)
         + bt_ref[...])
    x = jnp.maximum(x, 0.0)
    return (jnp.dot(x, o2w_ref[...], preferred_element_type=jnp.float32,
                precision=lax.Precision.HIGHEST)
            + o2b_ref[...])


def _node_body(s0_ref, s1_ref, cnt_ref, rt_ref, g0_ref, g1_ref, gb_ref,
               wr_ref, bt_ref, o2w_ref, o2b_ref, o_ref):
    o_ref[...] = _node_core(s0_ref, s1_ref, cnt_ref, rt_ref, g0_ref, g1_ref,
                            gb_ref, wr_ref, bt_ref, o2w_ref, o2b_ref)


def _node_final_body(s0_ref, s1_ref, cnt_ref, rt_ref, g0_ref, g1_ref, gb_ref,
                     wr_ref, bt_ref, o2w_ref, o2b_ref, ow1_ref, ob1_ref,
                     ow2_ref, o_ref):
    v = _node_core(s0_ref, s1_ref, cnt_ref, rt_ref, g0_ref, g1_ref,
                   gb_ref, wr_ref, bt_ref, o2w_ref, o2b_ref)
    y = jnp.maximum(
        jnp.dot(v, ow1_ref[...], preferred_element_type=jnp.float32,
                precision=lax.Precision.HIGHEST)
        + ob1_ref[...], 0.0)
    o_ref[...] = jnp.dot(y, ow2_ref[...], preferred_element_type=jnp.float32,
                precision=lax.Precision.HIGHEST)


_NODE_SPECS = [
    pl.BlockSpec((ROWS, HALF), lambda i: (i, 0)),
    pl.BlockSpec((ROWS, HALF), lambda i: (i, 0)),
    pl.BlockSpec((ROWS, HALF), lambda i: (i, 0)),
    pl.BlockSpec((ROWS, EMB), lambda i: (i, 0)),
    pl.BlockSpec((HALF, EMB), lambda i: (0, 0)),
    pl.BlockSpec((HALF, EMB), lambda i: (0, 0)),
    pl.BlockSpec((1, EMB), lambda i: (0, 0)),
    pl.BlockSpec((EMB, EMB), lambda i: (0, 0)),
    pl.BlockSpec((1, EMB), lambda i: (0, 0)),
    pl.BlockSpec((EMB, EMB), lambda i: (0, 0)),
    pl.BlockSpec((1, EMB), lambda i: (0, 0)),
]


def _node(s0, s1, cnt, right, g0, g1, gb, wr, bt, o2w, o2b):
    n = right.shape[0]
    g = n // ROWS
    return pl.pallas_call(
        _node_body,
        grid=(g,),
        in_specs=_NODE_SPECS,
        out_specs=pl.BlockSpec((ROWS, EMB), lambda i: (i, 0)),
        out_shape=jax.ShapeDtypeStruct((n, EMB), jnp.float32),
    )(s0, s1, cnt, right, g0, g1, gb.reshape(1, EMB), wr,
      bt.reshape(1, EMB), o2w, o2b.reshape(1, EMB))


def _node_final(s0, s1, cnt, right, g0, g1, gb, wr, bt, o2w, o2b,
                ow1, ob1, ow2p):
    n = right.shape[0]
    g = n // ROWS
    specs = _NODE_SPECS + [
        pl.BlockSpec((EMB, EMB), lambda i: (0, 0)),
        pl.BlockSpec((1, EMB), lambda i: (0, 0)),
        pl.BlockSpec((EMB, 8), lambda i: (0, 0)),
    ]
    return pl.pallas_call(
        _node_final_body,
        grid=(g,),
        in_specs=specs,
        out_specs=pl.BlockSpec((ROWS, 8), lambda i: (i, 0)),
        out_shape=jax.ShapeDtypeStruct((n, 8), jnp.float32),
    )(s0, s1, cnt, right, g0, g1, gb.reshape(1, EMB), wr,
      bt.reshape(1, EMB), o2w, o2b.reshape(1, EMB), ow1,
      ob1.reshape(1, EMB), ow2p)


@functools.lru_cache(maxsize=None)
def _make_edge_pass(n_nodes, n_edges):
    per_tile = n_edges // N_TILES
    n_blk = per_tile // BLK
    rows_per_tile = n_nodes // N_TILES

    mesh = plsc.VectorSubcoreMesh(core_axis_name="c", subcore_axis_name="s")

    @functools.partial(
        pl.kernel,
        mesh=mesh,
        compiler_params=pltpu.CompilerParams(use_tc_tiling_on_sc=False),
        out_type=[
            jax.ShapeDtypeStruct((n_nodes, HALF), jnp.float32),
            jax.ShapeDtypeStruct((n_nodes, HALF), jnp.float32),
            jax.ShapeDtypeStruct((n_nodes, HALF), jnp.float32),
        ],
        scratch_types=[
            pltpu.VMEM_SHARED((n_nodes, HALF), jnp.float32),
            pltpu.VMEM((BLK,), jnp.int32),
            pltpu.VMEM((BLK,), jnp.int32),
            pltpu.VMEM((BLK,), jnp.int32),
            pltpu.VMEM((2 * BLK,), jnp.float32),
            pltpu.VMEM((BLK, HALF), jnp.float32),
            pltpu.VMEM((BLK, HALF), jnp.float32),
            pltpu.VMEM((BLK, HALF), jnp.float32),
            pltpu.VMEM((BLK, HALF), jnp.float32),
            pltpu.VMEM((4, HALF), jnp.float32),
            pltpu.SemaphoreType.DMA,
            pltpu.SemaphoreType.DMA,
        ],
    )
    def edge_pass(a_hbm, b_hbm, ef_hbm, src_hbm, dst_hbm, par_hbm,
                  z32_hbm, one_hbm,
                  s0_out, s1_out, cnt_out,
                  s_sh, si_v, di_v, dg_v, ef_v, a_v, b_v, r_v,
                  one_v, par_v, sem_a, sem_b):
        cid = lax.axis_index("c")
        tid = lax.axis_index("s")

        @pl.when(tid == 0)
        def _():
            pltpu.sync_copy(z32_hbm, s_sh)

        pltpu.sync_copy(par_hbm.at[cid], par_v)
        pltpu.sync_copy(one_hbm, one_v)
        plsc.subcore_barrier()

        w = [par_v[0, pl.ds(h * LANES, LANES)] for h in range(2)]
        u = [par_v[1, pl.ds(h * LANES, LANES)] for h in range(2)]
        bb = [par_v[2, pl.ds(h * LANES, LANES)] for h in range(2)]
        fs = [par_v[3, pl.ds(h * LANES, LANES)] for h in range(2)]

        row_off = cid * n_nodes
        edge_base = tid * per_tile

        def body(i, carry):
            base = edge_base + i * BLK
            es = pl.ds(base, BLK)
            pltpu.sync_copy(src_hbm.at[es], si_v)
            pltpu.sync_copy(dst_hbm.at[es], di_v)
            pltpu.sync_copy(ef_hbm.at[pl.ds(base * 2, 2 * BLK)], ef_v)
            for g in range(BLK // LANES):
                gs = pl.ds(g * LANES, LANES)
                si_v[gs] = si_v[gs] + row_off
                dg_v[gs] = di_v[gs] + row_off
            ca = pltpu.async_copy(a_hbm.at[si_v], a_v, sem_a)
            cb = pltpu.async_copy(b_hbm.at[dg_v], b_v, sem_b)
            ca.wait()
            cb.wait()
            for e8 in range(BLK // 8):
                ev = ef_v[pl.ds(e8 * LANES, LANES)]
                for k in range(8):
                    e = e8 * 8 + k
                    s0 = ev[2 * k]
                    s1 = ev[2 * k + 1]
                    for h in range(2):
                        hs = pl.ds(h * LANES, LANES)
                        acc = (a_v[e, hs] + b_v[e, hs]
                               + w[h] * s0 + u[h] * s1 + bb[h])
                        r_v[e, hs] = jnp.maximum(acc * fs[h], 0.0)
            pltpu.sync_copy(r_v, s_sh.at[di_v], add=True)
            return carry

        lax.fori_loop(0, n_blk, body, 0)
        plsc.subcore_barrier()

        @pl.when(jnp.logical_and(cid == 0, tid == 0))
        def _():
            pltpu.sync_copy(s_sh, s0_out)

        @pl.when(jnp.logical_and(cid == 1, tid == 0))
        def _():
            pltpu.sync_copy(s_sh, s1_out)

        plsc.subcore_barrier()

        @pl.when(tid == 0)
        def _():
            pltpu.sync_copy(z32_hbm, s_sh)
        plsc.subcore_barrier()

        def cbody(i, carry):
            base = edge_base + i * BLK
            pltpu.sync_copy(dst_hbm.at[pl.ds(base, BLK)], di_v)
            pltpu.sync_copy(one_v, s_sh.at[di_v], add=True)
            return carry

        lax.fori_loop(0, n_blk, cbody, 0)
        plsc.subcore_barrier()

        @pl.when(jnp.logical_and(cid == 0, tid == 0))
        def _():
            pltpu.sync_copy(s_sh, cnt_out)

    return edge_pass


def _hdot(a, b):
    return jnp.dot(a, b, precision=lax.Precision.HIGHEST)


def _fold_in(shift, scale, w, b):
    wp = scale[:, None] * w
    bp = b - _hdot(shift * scale, w)
    return wp, bp


def _edge_par(e_w, e_b, conv_p):
    we = _hdot(e_w, conv_p['fe_W'])
    be = _hdot(e_b, conv_p['fe_W']) + conv_p['fl_b']
    rows = jnp.stack([we[0], we[1], be, conv_p['final_scale']])
    return jnp.stack([rows[:, :HALF], rows[:, HALF:]], axis=0)


def _node_folds(conv_p):
    wh = conv_p['pc_scale'][:, None] * conv_p['o1_W'][:EMB]
    bt = conv_p['o1_b'] - _hdot(
        conv_p['pc_shift'] * conv_p['pc_scale'], conv_p['o1_W'][:EMB])
    g0 = _hdot(conv_p['ff_W'][:HALF], wh)
    g1 = _hdot(conv_p['ff_W'][HALF:], wh)
    gb = _hdot(conv_p['ff_b'], wh)
    wr = conv_p['o1_W'][EMB:]
    return g0, g1, gb, wr, bt


def kernel(constraint_features, edge_indices, edge_features,
           variable_features, params):
    p = params
    n_cons = constraint_features.shape[0]
    n_vars = variable_features.shape[0]
    n_edges = edge_features.shape[0]

    cw1, cb1 = _fold_in(p['cons_shift'], p['cons_scale'], p['cons_W1'],
                        p['cons_b1'])
    vw1, vb1 = _fold_in(p['var_shift'], p['var_scale'], p['var_W1'],
                        p['var_b1'])
    c_emb = _mlp2(constraint_features, cw1, cb1, p['cons_W2'], p['cons_b2'])
    v_emb = _mlp2(variable_features, vw1, vb1, p['var_W2'], p['var_b2'])

    ew, eb = _fold_in(p['edge_shift'], p['edge_scale'], p['edge_W'],
                      p['edge_b'])
    par1 = _edge_par(ew, eb, p['v2c'])
    par2 = _edge_par(ew, eb, p['c2v'])

    src = edge_indices[0].astype(jnp.int32)
    dst = edge_indices[1].astype(jnp.int32)
    z32 = jnp.zeros((n_cons, HALF), jnp.float32)
    ones32 = jnp.ones((BLK, HALF), jnp.float32)

    edge_pass = _make_edge_pass(n_cons, n_edges)

    ef_flat = edge_features.reshape(-1)

    a1 = _matmul_packed(v_emb, p['v2c']['fl_W'])
    b1 = _matmul_packed(c_emb, p['v2c']['fr_W'])
    s0, s1, cnt = edge_pass(a1, b1, ef_flat, dst, src, par1,
                            z32, ones32)
    g0, g1, gb, wr, bt = _node_folds(p['v2c'])
    c_new = _node(s0, s1, cnt, c_emb, g0, g1, gb, wr, bt,
                  p['v2c']['o2_W'], p['v2c']['o2_b'])

    a2 = _matmul_packed(c_new, p['c2v']['fl_W'])
    b2 = _matmul_packed(v_emb, p['c2v']['fr_W'])
    t0, t1, vcnt = edge_pass(a2, b2, ef_flat, src, dst, par2,
                             z32, ones32)
    h0, h1, hb, hr, ht = _node_folds(p['c2v'])
    ow2p = jnp.concatenate(
        [p['out_W2'], jnp.zeros((EMB, 7), jnp.float32)], axis=1)
    out8 = _node_final(t0, t1, vcnt, v_emb, h0, h1, hb, hr, ht,
                       p['c2v']['o2_W'], p['c2v']['o2_b'],
                       p['out_W1'], p['out_b1'], ow2p)
    return out8[:, 0]
